# baseline (speedup 1.0000x reference)
import functools

import jax
import jax.numpy as jnp
import numpy as np
from jax import lax
from jax.experimental import pallas as pl
from jax.experimental.pallas import tpu as pltpu
from jax.experimental.pallas import tpu_sc as plsc

L = 16
NW = 32
TICK = 128
GROUPS = TICK // L


def _sqrt16(x):
    i = lax.bitcast_convert_type(x, jnp.int32)
    magic = jnp.full((L,), 0x5F3759DF, dtype=jnp.int32)
    r = lax.bitcast_convert_type(magic - lax.shift_right_logical(i, 1),
                                 jnp.float32)
    half = x * jnp.float32(0.5)
    c15 = jnp.full((L,), 1.5, dtype=jnp.float32)
    for _ in range(3):
        r = r * (c15 - half * r * r)
    return x * r


def _make_sc_kernel(n_idx, n_val_pad, t_per_w):
    groups1 = t_per_w // L
    cap = t_per_w + TICK + L
    mesh = plsc.VectorSubcoreMesh(core_axis_name="c", subcore_axis_name="s")

    @functools.partial(
        pl.kernel,
        mesh=mesh,
        compiler_params=pltpu.CompilerParams(
            needs_layout_passes=False, use_tc_tiling_on_sc=False),
        out_type=jax.ShapeDtypeStruct((NW, 2, L), jnp.float32),
        scratch_types=[
            pltpu.VMEM((n_idx,), jnp.int32),
            pltpu.VMEM((n_val_pad,), jnp.int32),
            pltpu.VMEM((t_per_w,), jnp.int32),
            pltpu.VMEM((t_per_w,), jnp.int32),
            pltpu.VMEM((t_per_w,), jnp.int32),
            pltpu.VMEM((cap,), jnp.int32),
            pltpu.VMEM((cap,), jnp.int32),
            pltpu.VMEM((cap,), jnp.int32),
            pltpu.VMEM((cap,), jnp.float32),
            pltpu.VMEM((2 * TICK, L), jnp.float32),
            pltpu.VMEM((2 * TICK, L), jnp.float32),
            pltpu.VMEM((2 * TICK, L), jnp.float32),
            pltpu.VMEM((2, L), jnp.float32),
            pltpu.VMEM_SHARED((n_idx, L), jnp.float32),
            pltpu.SemaphoreType.DMA,
            pltpu.SemaphoreType.DMA,
        ],
    )
    def sc_kernel(z_hbm, ind_hbm, val_hbm, a_hbm, p_hbm, n_hbm, out_hbm,
                  ind_v, val_v, av, pv, nv, ca, cp, cn, cm,
                  zab, zpb, znb, outst, zs, sem0, sem1):
        cid = lax.axis_index("c")
        sid = lax.axis_index("s")
        wid = cid * (NW // 2) + sid

        zrows = n_idx // (NW // 2)
        zsl = pl.ds(sid * zrows, zrows)
        z_stage = pltpu.make_async_copy(z_hbm.at[zsl], zs.at[zsl], sem1)
        z_stage.start()

        prologue = (
            pltpu.make_async_copy(ind_hbm, ind_v, sem0),
            pltpu.make_async_copy(val_hbm, val_v, sem0),
            pltpu.make_async_copy(a_hbm.at[wid], av, sem0),
            pltpu.make_async_copy(p_hbm.at[wid], pv, sem0),
            pltpu.make_async_copy(n_hbm.at[wid], nv, sem0),
        )
        for c in prologue:
            c.start()
        for c in prologue:
            c.wait()

        iota = lax.iota(jnp.int32, L)
        zero_f = jnp.zeros((L,), jnp.float32)
        zero_i = jnp.zeros((L,), jnp.int32)
        one_f = jnp.full((L,), 1.0, dtype=jnp.float32)

        @plsc.parallel_loop(0, groups1, carry=jnp.int32(0), unroll=16)
        def off(t, off):
            sl = pl.ds(t * L, L)
            a16 = av[sl]
            p16 = pv[sl]
            n16 = nv[sl]
            opa = plsc.load_gather(ind_v, [a16])
            opp = plsc.load_gather(ind_v, [p16])
            opn = plsc.load_gather(ind_v, [n16])
            vp = plsc.load_gather(val_v, [lax.abs(opa - opp)])
            vn = plsc.load_gather(val_v, [lax.abs(opa - opn)])
            valid = jnp.logical_and(
                vp > vn, jnp.logical_and(a16 != p16, a16 != n16))
            margin = (jnp.float32(0.1) +
                      jnp.float32(0.05) * lax.abs(vp - vn).astype(jnp.float32))
            dst = pl.ds(off, L)
            plsc.store_compressed(ca.at[dst], a16, mask=valid)
            plsc.store_compressed(cp.at[dst], p16, mask=valid)
            plsc.store_compressed(cn.at[dst], n16, mask=valid)
            plsc.store_compressed(cm.at[dst], margin, mask=valid)
            return off + jnp.sum(valid.astype(jnp.int32))

        for i in range(TICK // L):
            tsl = pl.ds(off + i * L, L)
            ca[tsl] = zero_i
            cp[tsl] = zero_i
            cn[tsl] = zero_i
            cm[tsl] = zero_f

        nt2 = lax.div(off + TICK - 1, jnp.int32(TICK))

        z_stage.wait()
        plsc.subcore_barrier()

        def copies(j, buf, sem):
            src = pl.ds(j * TICK, TICK)
            dst = pl.ds(buf * TICK, TICK)
            return (
                pltpu.make_async_copy(zs.at[ca.at[src]], zab.at[dst], sem),
                pltpu.make_async_copy(zs.at[cp.at[src]], zpb.at[dst], sem),
                pltpu.make_async_copy(zs.at[cn.at[src]], znb.at[dst], sem),
            )

        def fire(j, buf, sem):
            for c in copies(j, buf, sem):
                c.start()

        def drain(j, buf, sem):
            for c in copies(j, buf, sem):
                c.wait()

        @pl.when(nt2 > 0)
        def _():
            fire(0, 0, sem0)

        def body2(j, sum_acc):
            b = lax.rem(j, 2)

            @pl.when(jnp.logical_and(b == 0, j + 1 < nt2))
            def _():
                fire(j + 1, 1, sem1)

            @pl.when(jnp.logical_and(b == 1, j + 1 < nt2))
            def _():
                fire(j + 1, 0, sem0)

            @pl.when(b == 0)
            def _():
                drain(j, 0, sem0)

            @pl.when(b == 1)
            def _():
                drain(j, 1, sem1)

            row0 = b * TICK

            @plsc.parallel_loop(0, GROUPS, carry=sum_acc, unroll=8)
            def sum_acc(g, acc):
                rows = row0 + g * L + iota
                accp = zero_f
                accn = zero_f
                for d in range(L):
                    dcol = lax.bitwise_and(iota + d, jnp.int32(L - 1))
                    za = plsc.load_gather(zab, [rows, dcol])
                    zp = plsc.load_gather(zpb, [rows, dcol])
                    zn = plsc.load_gather(znb, [rows, dcol])
                    tp = za - zp
                    accp = accp + tp * tp
                    tn = za - zn
                    accn = accn + tn * tn

                dlp = _sqrt16(accp)
                dln = _sqrt16(accn)
                m16 = cm[pl.ds(j * TICK + g * L, L)]
                per = jnp.maximum(dlp - dln + m16, zero_f)
                return acc + per
            return sum_acc

        sum_acc = lax.fori_loop(0, nt2, body2, zero_f)

        outst[0, :] = sum_acc
        outst[1, :] = jnp.full((L,), 1.0, jnp.float32) * off.astype(jnp.float32)
        pltpu.sync_copy(outst, out_hbm.at[wid])

    return sc_kernel


_TRIPLET_CACHE = {}


def _triplet_ids(batch, n):
    if (batch, n) not in _TRIPLET_CACHE:
        with jax.ensure_compile_time_eval():
            key = jax.random.key(42)
            ka, kp, kn = jax.random.split(key, 3)
            anchor = jax.random.randint(ka, (n,), 0, batch)
            pos = jax.random.randint(kp, (n,), 0, batch)
            neg = jax.random.randint(kn, (n,), 0, batch)
        chunk = NW * TICK
        n_pad = ((n + chunk - 1) // chunk) * chunk
        t_per_w = n_pad // NW

        def prep(x):
            x = np.concatenate(
                [np.asarray(x, np.int32), np.zeros((n_pad - n,), np.int32)])
            return x.reshape(NW, t_per_w)

        _TRIPLET_CACHE[batch, n] = (prep(anchor), prep(pos), prep(neg))
    return _TRIPLET_CACHE[batch, n]


def kernel(z, indices, valuations):
    batch = z.shape[0]
    n = min(100000, batch * (batch - 1) * (batch - 2) // 6)
    a_np, p_np, n_np = _triplet_ids(batch, n)
    t_per_w = a_np.shape[1]
    a_arr = jnp.asarray(a_np)
    p_arr = jnp.asarray(p_np)
    n_arr = jnp.asarray(n_np)

    n_val = valuations.shape[0]
    n_val_pad = ((n_val + 127) // 128) * 128
    val_pad = jnp.concatenate(
        [valuations.astype(jnp.int32),
         jnp.zeros((n_val_pad - n_val,), jnp.int32)])

    sc_kernel = _make_sc_kernel(indices.shape[0], n_val_pad, t_per_w)
    parts = sc_kernel(z, indices.astype(jnp.int32), val_pad,
                      a_arr, p_arr, n_arr)

    total = jnp.sum(parts[:, 0, :])
    cnt = jnp.sum(parts[:, 1, 0])
    return jnp.where(cnt > 0, total / jnp.maximum(cnt, 1.0), 0.0)


# speedup vs baseline: 1.0760x; 1.0760x over previous
import functools

import jax
import jax.numpy as jnp
import numpy as np
from jax import lax
from jax.experimental import pallas as pl
from jax.experimental.pallas import tpu as pltpu
from jax.experimental.pallas import tpu_sc as plsc

L = 16
NW = 32
TICK = 128
GROUPS = TICK // L


def _sqrt16(x):
    i = lax.bitcast_convert_type(x, jnp.int32)
    magic = jnp.full((L,), 0x5F3759DF, dtype=jnp.int32)
    r = lax.bitcast_convert_type(magic - lax.shift_right_logical(i, 1),
                                 jnp.float32)
    half = x * jnp.float32(0.5)
    c15 = jnp.full((L,), 1.5, dtype=jnp.float32)
    for _ in range(3):
        r = r * (c15 - half * r * r)
    return x * r


def _make_sc_kernel(n_idx, n_val_pad, t_per_w):
    groups1 = t_per_w // L
    cap = t_per_w + TICK + L
    mesh = plsc.VectorSubcoreMesh(core_axis_name="c", subcore_axis_name="s")

    @functools.partial(
        pl.kernel,
        mesh=mesh,
        compiler_params=pltpu.CompilerParams(
            needs_layout_passes=False, use_tc_tiling_on_sc=False),
        out_type=jax.ShapeDtypeStruct((NW, 2, L), jnp.float32),
        scratch_types=[
            pltpu.VMEM((n_idx,), jnp.int32),
            pltpu.VMEM((n_val_pad,), jnp.int32),
            pltpu.VMEM((t_per_w,), jnp.int32),
            pltpu.VMEM((t_per_w,), jnp.int32),
            pltpu.VMEM((t_per_w,), jnp.int32),
            pltpu.VMEM((cap,), jnp.int32),
            pltpu.VMEM((cap,), jnp.int32),
            pltpu.VMEM((cap,), jnp.int32),
            pltpu.VMEM((cap,), jnp.float32),
            pltpu.VMEM((2 * TICK, L), jnp.float32),
            pltpu.VMEM((2 * TICK, L), jnp.float32),
            pltpu.VMEM((2 * TICK, L), jnp.float32),
            pltpu.VMEM((2, L), jnp.float32),
            pltpu.VMEM_SHARED((n_idx, L), jnp.float32),
            pltpu.SemaphoreType.DMA,
            pltpu.SemaphoreType.DMA,
        ],
    )
    def sc_kernel(z_hbm, ind_hbm, val_hbm, a_hbm, p_hbm, n_hbm, out_hbm,
                  ind_v, val_v, av, pv, nv, ca, cp, cn, cm,
                  zab, zpb, znb, outst, zs, sem0, sem1):
        cid = lax.axis_index("c")
        sid = lax.axis_index("s")
        wid = cid * (NW // 2) + sid

        zrows = n_idx // (NW // 2)
        zsl = pl.ds(sid * zrows, zrows)
        z_stage = pltpu.make_async_copy(z_hbm.at[zsl], zs.at[zsl], sem1)
        z_stage.start()

        prologue = (
            pltpu.make_async_copy(ind_hbm, ind_v, sem0),
            pltpu.make_async_copy(val_hbm, val_v, sem0),
            pltpu.make_async_copy(a_hbm.at[wid], av, sem0),
            pltpu.make_async_copy(p_hbm.at[wid], pv, sem0),
            pltpu.make_async_copy(n_hbm.at[wid], nv, sem0),
        )
        for c in prologue:
            c.start()
        for c in prologue:
            c.wait()

        iota = lax.iota(jnp.int32, L)
        zero_f = jnp.zeros((L,), jnp.float32)
        zero_i = jnp.zeros((L,), jnp.int32)
        one_f = jnp.full((L,), 1.0, dtype=jnp.float32)

        @plsc.parallel_loop(0, groups1, carry=jnp.int32(0), unroll=8)
        def off(t, off):
            sl = pl.ds(t * L, L)
            a16 = av[sl]
            p16 = pv[sl]
            n16 = nv[sl]
            opa = plsc.load_gather(ind_v, [a16])
            opp = plsc.load_gather(ind_v, [p16])
            opn = plsc.load_gather(ind_v, [n16])
            vp = plsc.load_gather(val_v, [lax.abs(opa - opp)])
            vn = plsc.load_gather(val_v, [lax.abs(opa - opn)])
            valid = jnp.logical_and(
                vp > vn, jnp.logical_and(a16 != p16, a16 != n16))
            margin = (jnp.float32(0.1) +
                      jnp.float32(0.05) * lax.abs(vp - vn).astype(jnp.float32))
            dst = pl.ds(off, L)
            plsc.store_compressed(ca.at[dst], a16, mask=valid)
            plsc.store_compressed(cp.at[dst], p16, mask=valid)
            plsc.store_compressed(cn.at[dst], n16, mask=valid)
            plsc.store_compressed(cm.at[dst], margin, mask=valid)
            return off + jnp.sum(valid.astype(jnp.int32))

        for i in range(TICK // L):
            tsl = pl.ds(off + i * L, L)
            ca[tsl] = zero_i
            cp[tsl] = zero_i
            cn[tsl] = zero_i
            cm[tsl] = zero_f

        nt2 = lax.div(off + TICK - 1, jnp.int32(TICK))

        z_stage.wait()
        plsc.subcore_barrier()

        def copies(j, buf, sem):
            src = pl.ds(j * TICK, TICK)
            dst = pl.ds(buf * TICK, TICK)
            return (
                pltpu.make_async_copy(zs.at[ca.at[src]], zab.at[dst], sem),
                pltpu.make_async_copy(zs.at[cp.at[src]], zpb.at[dst], sem),
                pltpu.make_async_copy(zs.at[cn.at[src]], znb.at[dst], sem),
            )

        def fire(j, buf, sem):
            for c in copies(j, buf, sem):
                c.start()

        def drain(j, buf, sem):
            for c in copies(j, buf, sem):
                c.wait()

        @pl.when(nt2 > 0)
        def _():
            fire(0, 0, sem0)

        def body2(j, sum_acc):
            b = lax.rem(j, 2)

            @pl.when(jnp.logical_and(b == 0, j + 1 < nt2))
            def _():
                fire(j + 1, 1, sem1)

            @pl.when(jnp.logical_and(b == 1, j + 1 < nt2))
            def _():
                fire(j + 1, 0, sem0)

            @pl.when(b == 0)
            def _():
                drain(j, 0, sem0)

            @pl.when(b == 1)
            def _():
                drain(j, 1, sem1)

            row0 = b * TICK

            @plsc.parallel_loop(0, GROUPS, carry=sum_acc, unroll=8)
            def sum_acc(g, acc):
                rows = row0 + g * L + iota
                accp = zero_f
                accn = zero_f
                for d in range(L):
                    dcol = lax.bitwise_and(iota + d, jnp.int32(L - 1))
                    za = plsc.load_gather(zab, [rows, dcol])
                    zp = plsc.load_gather(zpb, [rows, dcol])
                    zn = plsc.load_gather(znb, [rows, dcol])
                    tp = za - zp
                    accp = accp + tp * tp
                    tn = za - zn
                    accn = accn + tn * tn

                dlp = _sqrt16(accp)
                dln = _sqrt16(accn)
                m16 = cm[pl.ds(j * TICK + g * L, L)]
                per = jnp.maximum(dlp - dln + m16, zero_f)
                return acc + per
            return sum_acc

        sum_acc = lax.fori_loop(0, nt2, body2, zero_f)

        outst[0, :] = sum_acc
        outst[1, :] = jnp.full((L,), 1.0, jnp.float32) * off.astype(jnp.float32)
        pltpu.sync_copy(outst, out_hbm.at[wid])

    return sc_kernel


_TRIPLET_CACHE = {}


def _triplet_ids(batch, n):
    if (batch, n) not in _TRIPLET_CACHE:
        with jax.ensure_compile_time_eval():
            key = jax.random.key(42)
            ka, kp, kn = jax.random.split(key, 3)
            anchor = jax.random.randint(ka, (n,), 0, batch)
            pos = jax.random.randint(kp, (n,), 0, batch)
            neg = jax.random.randint(kn, (n,), 0, batch)
        chunk = NW * TICK
        n_pad = ((n + chunk - 1) // chunk) * chunk
        t_per_w = n_pad // NW

        def prep(x):
            x = np.concatenate(
                [np.asarray(x, np.int32), np.zeros((n_pad - n,), np.int32)])
            return x.reshape(NW, t_per_w)

        _TRIPLET_CACHE[batch, n] = (prep(anchor), prep(pos), prep(neg))
    return _TRIPLET_CACHE[batch, n]


def kernel(z, indices, valuations):
    batch = z.shape[0]
    n = min(100000, batch * (batch - 1) * (batch - 2) // 6)
    a_np, p_np, n_np = _triplet_ids(batch, n)
    t_per_w = a_np.shape[1]
    a_arr = jnp.asarray(a_np)
    p_arr = jnp.asarray(p_np)
    n_arr = jnp.asarray(n_np)

    n_val = valuations.shape[0]
    n_val_pad = ((n_val + 127) // 128) * 128
    val_pad = jnp.concatenate(
        [valuations.astype(jnp.int32),
         jnp.zeros((n_val_pad - n_val,), jnp.int32)])

    sc_kernel = _make_sc_kernel(indices.shape[0], n_val_pad, t_per_w)
    parts = sc_kernel(z, indices.astype(jnp.int32), val_pad,
                      a_arr, p_arr, n_arr)

    total = jnp.sum(parts[:, 0, :])
    cnt = jnp.sum(parts[:, 1, 0])
    return jnp.where(cnt > 0, total / jnp.maximum(cnt, 1.0), 0.0)


# speedup vs baseline: 1.0789x; 1.0027x over previous
import functools

import jax
import jax.numpy as jnp
import numpy as np
from jax import lax
from jax.experimental import pallas as pl
from jax.experimental.pallas import tpu as pltpu
from jax.experimental.pallas import tpu_sc as plsc

L = 16
NW = 32
TICK = 128
GROUPS = TICK // L


def _sqrt16(x):
    i = lax.bitcast_convert_type(x, jnp.int32)
    magic = jnp.full((L,), 0x5F3759DF, dtype=jnp.int32)
    r = lax.bitcast_convert_type(magic - lax.shift_right_logical(i, 1),
                                 jnp.float32)
    half = x * jnp.float32(0.5)
    c15 = jnp.full((L,), 1.5, dtype=jnp.float32)
    for _ in range(3):
        r = r * (c15 - half * r * r)
    return x * r


def _make_sc_kernel(n_idx, n_val_pad, t_per_w):
    groups1 = t_per_w // L
    cap = t_per_w + TICK + L
    mesh = plsc.VectorSubcoreMesh(core_axis_name="c", subcore_axis_name="s")

    @functools.partial(
        pl.kernel,
        mesh=mesh,
        compiler_params=pltpu.CompilerParams(
            needs_layout_passes=False, use_tc_tiling_on_sc=False),
        out_type=jax.ShapeDtypeStruct((NW, 2, L), jnp.float32),
        scratch_types=[
            pltpu.VMEM((n_idx,), jnp.int32),
            pltpu.VMEM((n_val_pad,), jnp.int32),
            pltpu.VMEM((t_per_w,), jnp.int32),
            pltpu.VMEM((t_per_w,), jnp.int32),
            pltpu.VMEM((t_per_w,), jnp.int32),
            pltpu.VMEM((cap,), jnp.int32),
            pltpu.VMEM((cap,), jnp.int32),
            pltpu.VMEM((cap,), jnp.int32),
            pltpu.VMEM((cap,), jnp.float32),
            pltpu.VMEM((2 * TICK, L), jnp.float32),
            pltpu.VMEM((2 * TICK, L), jnp.float32),
            pltpu.VMEM((2 * TICK, L), jnp.float32),
            pltpu.VMEM((2, L), jnp.float32),
            pltpu.VMEM_SHARED((n_idx, L), jnp.float32),
            pltpu.SemaphoreType.DMA,
            pltpu.SemaphoreType.DMA,
        ],
    )
    def sc_kernel(z_hbm, ind_hbm, val_hbm, a_hbm, p_hbm, n_hbm, out_hbm,
                  ind_v, val_v, av, pv, nv, ca, cp, cn, cm,
                  zab, zpb, znb, outst, zs, sem0, sem1):
        cid = lax.axis_index("c")
        sid = lax.axis_index("s")
        wid = cid * (NW // 2) + sid

        zrows = n_idx // (NW // 2)
        zsl = pl.ds(sid * zrows, zrows)
        z_stage = pltpu.make_async_copy(z_hbm.at[zsl], zs.at[zsl], sem1)
        z_stage.start()

        prologue = (
            pltpu.make_async_copy(ind_hbm, ind_v, sem0),
            pltpu.make_async_copy(val_hbm, val_v, sem0),
            pltpu.make_async_copy(a_hbm.at[wid], av, sem0),
            pltpu.make_async_copy(p_hbm.at[wid], pv, sem0),
            pltpu.make_async_copy(n_hbm.at[wid], nv, sem0),
        )
        for c in prologue:
            c.start()
        for c in prologue:
            c.wait()

        iota = lax.iota(jnp.int32, L)
        zero_f = jnp.zeros((L,), jnp.float32)
        zero_i = jnp.zeros((L,), jnp.int32)
        one_f = jnp.full((L,), 1.0, dtype=jnp.float32)

        @plsc.parallel_loop(0, groups1, carry=jnp.int32(0), unroll=8)
        def off(t, off):
            sl = pl.ds(t * L, L)
            a16 = av[sl]
            p16 = pv[sl]
            n16 = nv[sl]
            opa = plsc.load_gather(ind_v, [a16])
            opp = plsc.load_gather(ind_v, [p16])
            opn = plsc.load_gather(ind_v, [n16])
            vp = plsc.load_gather(val_v, [lax.abs(opa - opp)])
            vn = plsc.load_gather(val_v, [lax.abs(opa - opn)])
            valid = jnp.logical_and(
                vp > vn, jnp.logical_and(a16 != p16, a16 != n16))
            margin = (jnp.float32(0.1) +
                      jnp.float32(0.05) * lax.abs(vp - vn).astype(jnp.float32))
            dst = pl.ds(off, L)
            plsc.store_compressed(ca.at[dst], a16, mask=valid)
            plsc.store_compressed(cp.at[dst], p16, mask=valid)
            plsc.store_compressed(cn.at[dst], n16, mask=valid)
            plsc.store_compressed(cm.at[dst], margin, mask=valid)
            return off + jnp.sum(valid.astype(jnp.int32))

        for i in range(TICK // L):
            tsl = pl.ds(off + i * L, L)
            ca[tsl] = zero_i
            cp[tsl] = zero_i
            cn[tsl] = zero_i
            cm[tsl] = zero_f

        nt2 = lax.div(off + TICK - 1, jnp.int32(TICK))

        z_stage.wait()
        plsc.subcore_barrier()

        def copies(j, buf, sem):
            src = pl.ds(j * TICK, TICK)
            dst = pl.ds(buf * TICK, TICK)
            return (
                pltpu.make_async_copy(zs.at[ca.at[src]], zab.at[dst], sem),
                pltpu.make_async_copy(zs.at[cp.at[src]], zpb.at[dst], sem),
                pltpu.make_async_copy(zs.at[cn.at[src]], znb.at[dst], sem),
            )

        def fire(j, buf, sem):
            for c in copies(j, buf, sem):
                c.start()

        def drain(j, buf, sem):
            for c in copies(j, buf, sem):
                c.wait()

        @pl.when(nt2 > 0)
        def _():
            fire(0, 0, sem0)

        def body2(j, sum_acc):
            b = lax.rem(j, 2)

            @pl.when(jnp.logical_and(b == 0, j + 1 < nt2))
            def _():
                fire(j + 1, 1, sem1)

            @pl.when(jnp.logical_and(b == 1, j + 1 < nt2))
            def _():
                fire(j + 1, 0, sem0)

            @pl.when(b == 0)
            def _():
                drain(j, 0, sem0)

            @pl.when(b == 1)
            def _():
                drain(j, 1, sem1)

            row0 = b * TICK

            @plsc.parallel_loop(0, GROUPS, carry=sum_acc, unroll=4)
            def sum_acc(g, acc):
                rows = row0 + g * L + iota
                accp = zero_f
                accn = zero_f
                for d in range(L):
                    dcol = lax.bitwise_and(iota + d, jnp.int32(L - 1))
                    za = plsc.load_gather(zab, [rows, dcol])
                    zp = plsc.load_gather(zpb, [rows, dcol])
                    zn = plsc.load_gather(znb, [rows, dcol])
                    tp = za - zp
                    accp = accp + tp * tp
                    tn = za - zn
                    accn = accn + tn * tn

                dlp = _sqrt16(accp)
                dln = _sqrt16(accn)
                m16 = cm[pl.ds(j * TICK + g * L, L)]
                per = jnp.maximum(dlp - dln + m16, zero_f)
                return acc + per
            return sum_acc

        sum_acc = lax.fori_loop(0, nt2, body2, zero_f)

        outst[0, :] = sum_acc
        outst[1, :] = jnp.full((L,), 1.0, jnp.float32) * off.astype(jnp.float32)
        pltpu.sync_copy(outst, out_hbm.at[wid])

    return sc_kernel


_TRIPLET_CACHE = {}


def _triplet_ids(batch, n):
    if (batch, n) not in _TRIPLET_CACHE:
        with jax.ensure_compile_time_eval():
            key = jax.random.key(42)
            ka, kp, kn = jax.random.split(key, 3)
            anchor = jax.random.randint(ka, (n,), 0, batch)
            pos = jax.random.randint(kp, (n,), 0, batch)
            neg = jax.random.randint(kn, (n,), 0, batch)
        chunk = NW * TICK
        n_pad = ((n + chunk - 1) // chunk) * chunk
        t_per_w = n_pad // NW

        def prep(x):
            x = np.concatenate(
                [np.asarray(x, np.int32), np.zeros((n_pad - n,), np.int32)])
            return x.reshape(NW, t_per_w)

        _TRIPLET_CACHE[batch, n] = (prep(anchor), prep(pos), prep(neg))
    return _TRIPLET_CACHE[batch, n]


def kernel(z, indices, valuations):
    batch = z.shape[0]
    n = min(100000, batch * (batch - 1) * (batch - 2) // 6)
    a_np, p_np, n_np = _triplet_ids(batch, n)
    t_per_w = a_np.shape[1]
    a_arr = jnp.asarray(a_np)
    p_arr = jnp.asarray(p_np)
    n_arr = jnp.asarray(n_np)

    n_val = valuations.shape[0]
    n_val_pad = ((n_val + 127) // 128) * 128
    val_pad = jnp.concatenate(
        [valuations.astype(jnp.int32),
         jnp.zeros((n_val_pad - n_val,), jnp.int32)])

    sc_kernel = _make_sc_kernel(indices.shape[0], n_val_pad, t_per_w)
    parts = sc_kernel(z, indices.astype(jnp.int32), val_pad,
                      a_arr, p_arr, n_arr)

    total = jnp.sum(parts[:, 0, :])
    cnt = jnp.sum(parts[:, 1, 0])
    return jnp.where(cnt > 0, total / jnp.maximum(cnt, 1.0), 0.0)
